# baseline (speedup 1.0000x reference)
import jax
import jax.numpy as jnp
from jax.experimental import pallas as pl
from jax.experimental.pallas import tpu as pltpu

N = 33792
G = 128
NPG = 264
K = 132
H = 128
IN = 263
E = 270336
HC = 512
FLAT = K * H
EL = E + N
MB = 512
NBLK = N // MB


def _conv0_mm_kernel(x_ref, w_ref, as_ref, ad_ref, h_ref, hs_ref, hd_ref):
    h = jax.lax.dot_general(x_ref[...], w_ref[...], (((1,), (0,)), ((), ())),
                            preferred_element_type=jnp.float32)
    h_ref[...] = h
    hs_ref[...] = jax.lax.dot_general(h, as_ref[...], (((1,), (0,)), ((), ())),
                                      preferred_element_type=jnp.float32)
    hd_ref[...] = jax.lax.dot_general(h, ad_ref[...], (((1,), (0,)), ((), ())),
                                      preferred_element_type=jnp.float32)


def _conv12_mm_kernel(agg_ref, b_ref, w1_ref, as1_ref, ad1_ref,
                      w2_ref, as2_ref, ad2_ref,
                      h1_ref, hs1_ref, hd1_ref, h2_ref, hs2_ref, hd2_ref):
    x0 = jnp.maximum(agg_ref[...] + b_ref[...], 0.0)
    h1 = jax.lax.dot_general(x0, w1_ref[...], (((1,), (0,)), ((), ())),
                             preferred_element_type=jnp.float32)
    h1_ref[...] = h1
    hs1_ref[...] = jax.lax.dot_general(h1, as1_ref[...], (((1,), (0,)), ((), ())),
                                       preferred_element_type=jnp.float32)
    hd1_ref[...] = jax.lax.dot_general(h1, ad1_ref[...], (((1,), (0,)), ((), ())),
                                       preferred_element_type=jnp.float32)
    h2 = jax.lax.dot_general(x0, w2_ref[...], (((1,), (0,)), ((), ())),
                             preferred_element_type=jnp.float32)
    h2_ref[...] = h2
    hs2_ref[...] = jax.lax.dot_general(h2, as2_ref[...], (((1,), (0,)), ((), ())),
                                       preferred_element_type=jnp.float32)
    hd2_ref[...] = jax.lax.dot_general(h2, ad2_ref[...], (((1,), (0,)), ((), ())),
                                       preferred_element_type=jnp.float32)


def _final_act_kernel(agg1_ref, b1_ref, agg2_ref, b2_ref, p_ref,
                      x11_ref, x22_ref, z11_ref, z22_ref):
    x11 = jnp.maximum(agg1_ref[...] + b1_ref[...], 0.0)
    x22 = jnp.maximum(agg2_ref[...] + b2_ref[...], 0.0)
    x11_ref[...] = x11
    x22_ref[...] = x22
    for i in range(8):
        p_col = p_ref[:, i:i + 1]
        z11_ref[:, i:i + 1] = jax.lax.dot_general(
            x11, p_col, (((1,), (0,)), ((), ())), preferred_element_type=jnp.float32)
        z22_ref[:, i:i + 1] = jax.lax.dot_general(
            x22, p_col, (((1,), (0,)), ((), ())), preferred_element_type=jnp.float32)


def _edge_e_kernel(hss_ref, hdd_ref, e_ref):
    e_ref[...] = jax.nn.leaky_relu(hss_ref[...] + hdd_ref[...], 0.2)


def _edge_ex_kernel(e_ref, mdst_ref, ex_ref):
    ex_ref[...] = jnp.exp(e_ref[...] - mdst_ref[...])


def _edge_alpha_kernel(ex_ref, sdst_ref, a_ref):
    a_ref[...] = ex_ref[...] / (sdst_ref[...] + 1e-16)


def _pools_kernel(x11_ref, x22_ref, z11_ref, z11t_ref, z22_ref, z22t_ref, nrm_ref,
                  *out_refs):
    g = pl.program_id(0)
    kio_c = jax.lax.broadcasted_iota(jnp.int32, (K, 1), 0).astype(jnp.float32)
    kio_r = jax.lax.broadcasted_iota(jnp.int32, (1, K), 1).astype(jnp.float32)
    jio_c = jax.lax.broadcasted_iota(jnp.int32, (NPG, 1), 0).astype(jnp.float32)
    jio_r = jax.lax.broadcasted_iota(jnp.int32, (1, NPG), 1).astype(jnp.float32)
    ones_r = jnp.ones((1, NPG), jnp.float32)
    ones_c = jnp.ones((NPG, 1), jnp.float32)
    for i in range(8):
        if i < 4:
            xin = x11_ref[0]
            u_c = z11_ref[0, :, i:i + 1]
            u_r = z11t_ref[0, i:i + 1, :]
        else:
            xin = x22_ref[0]
            u_c = z22_ref[0, :, i:i + 1]
            u_r = z22t_ref[0, i:i + 1, :]
        d = nrm_ref[0:1, i:i + 1]
        s_c = jax.nn.sigmoid(u_c / d)
        s_r = jax.nn.sigmoid(u_r / d)
        cmp1 = jnp.where((s_c > s_r) | ((s_c == s_r) & (jio_c < jio_r)), 1.0, 0.0)
        rank_r = jax.lax.dot_general(ones_r, cmp1, (((1,), (0,)), ((), ())),
                                     preferred_element_type=jnp.float32)
        cmp2 = jnp.where((s_r > s_c) | ((s_r == s_c) & (jio_r < jio_c)), 1.0, 0.0)
        rank_c = jax.lax.dot_general(cmp2, ones_c, (((1,), (0,)), ((), ())),
                                     preferred_element_type=jnp.float32)
        onehot = jnp.where(kio_c == rank_r, 1.0, 0.0)
        onehot_t = jnp.where(rank_c == kio_r, 1.0, 0.0)
        idx_r = jax.lax.dot_general(jio_r, onehot_t, (((1,), (0,)), ((), ())),
                                    preferred_element_type=jnp.float32)
        vals_r = jax.lax.dot_general(s_r, onehot_t, (((1,), (0,)), ((), ())),
                                     preferred_element_type=jnp.float32)
        xp = jax.lax.dot_general(onehot, s_c * xin, (((1,), (0,)), ((), ())),
                                 preferred_element_type=jnp.float32)
        perm = (idx_r + (g * NPG).astype(jnp.float32)).astype(jnp.int32)
        sig = jax.nn.sigmoid(vals_r)
        out_refs[3 * i][...] = xp.reshape(1, K, H)
        out_refs[3 * i + 1][...] = perm.reshape(1, 1, K)
        out_refs[3 * i + 2][...] = sig.reshape(1, 1, K)


def _head_kernel(f_ref, w1_ref, b1_ref, w3_ref, b3_ref, o_ref, acc_ref):
    kc = pl.program_id(0)

    @pl.when(kc == 0)
    def _():
        acc_ref[...] = jnp.zeros_like(acc_ref)

    acc_ref[...] += jax.lax.dot_general(
        jnp.maximum(f_ref[...], 0.0), w1_ref[...], (((1,), (0,)), ((), ())),
        preferred_element_type=jnp.float32)

    @pl.when(kc == pl.num_programs(0) - 1)
    def _():
        h1 = jnp.maximum(acc_ref[...] + b1_ref[...], 0.0)
        o_ref[...] = jax.lax.dot_general(h1, w3_ref[...], (((1,), (0,)), ((), ())),
                                         preferred_element_type=jnp.float32) + b3_ref[...]


def _conv0_mm(x, W, a_s, a_d):
    return pl.pallas_call(
        _conv0_mm_kernel,
        grid=(NBLK,),
        in_specs=[pl.BlockSpec((MB, IN), lambda i: (i, 0)),
                  pl.BlockSpec((IN, H), lambda i: (0, 0)),
                  pl.BlockSpec((H, 1), lambda i: (0, 0)),
                  pl.BlockSpec((H, 1), lambda i: (0, 0))],
        out_specs=[pl.BlockSpec((MB, H), lambda i: (i, 0)),
                   pl.BlockSpec((MB, 1), lambda i: (i, 0)),
                   pl.BlockSpec((MB, 1), lambda i: (i, 0))],
        out_shape=[jax.ShapeDtypeStruct((N, H), jnp.float32),
                   jax.ShapeDtypeStruct((N, 1), jnp.float32),
                   jax.ShapeDtypeStruct((N, 1), jnp.float32)],
    )(x, W, a_s.reshape(H, 1), a_d.reshape(H, 1))


def _conv12_mm(agg0, b0, W1, as1, ad1, W2, as2, ad2):
    w_spec = pl.BlockSpec((H, H), lambda i: (0, 0))
    v_spec = pl.BlockSpec((H, 1), lambda i: (0, 0))
    m_spec = pl.BlockSpec((MB, H), lambda i: (i, 0))
    c_spec = pl.BlockSpec((MB, 1), lambda i: (i, 0))
    mat = jax.ShapeDtypeStruct((N, H), jnp.float32)
    col = jax.ShapeDtypeStruct((N, 1), jnp.float32)
    return pl.pallas_call(
        _conv12_mm_kernel,
        grid=(NBLK,),
        in_specs=[m_spec, pl.BlockSpec((1, H), lambda i: (0, 0)),
                  w_spec, v_spec, v_spec, w_spec, v_spec, v_spec],
        out_specs=[m_spec, c_spec, c_spec, m_spec, c_spec, c_spec],
        out_shape=[mat, col, col, mat, col, col],
    )(agg0, b0.reshape(1, H), W1, as1.reshape(H, 1), ad1.reshape(H, 1),
      W2, as2.reshape(H, 1), ad2.reshape(H, 1))


def _final_act(agg1, b1, agg2, b2, pools):
    m_spec = pl.BlockSpec((MB, H), lambda i: (i, 0))
    z_spec = pl.BlockSpec((MB, 8), lambda i: (i, 0))
    return pl.pallas_call(
        _final_act_kernel,
        grid=(NBLK,),
        in_specs=[m_spec, pl.BlockSpec((1, H), lambda i: (0, 0)),
                  m_spec, pl.BlockSpec((1, H), lambda i: (0, 0)),
                  pl.BlockSpec((H, 8), lambda i: (0, 0))],
        out_specs=[m_spec, m_spec, z_spec, z_spec],
        out_shape=[jax.ShapeDtypeStruct((N, H), jnp.float32),
                   jax.ShapeDtypeStruct((N, H), jnp.float32),
                   jax.ShapeDtypeStruct((N, 8), jnp.float32),
                   jax.ShapeDtypeStruct((N, 8), jnp.float32)],
    )(agg1, b1.reshape(1, H), agg2, b2.reshape(1, H), pools)


_EROWS = EL // 128


def _edge_map(kernel, *arrs):
    arrs2 = [a.reshape(_EROWS, 128) for a in arrs]
    out = pl.pallas_call(
        kernel,
        out_shape=jax.ShapeDtypeStruct((_EROWS, 128), jnp.float32),
    )(*arrs2)
    return out.reshape(EL)


def _pools(x11, x22, z11, z22, norms):
    x11g = x11.reshape(G, NPG, H)
    x22g = x22.reshape(G, NPG, H)
    z11g = z11.reshape(G, NPG, 8)
    z22g = z22.reshape(G, NPG, 8)
    z11t = jnp.transpose(z11g, (0, 2, 1))
    z22t = jnp.transpose(z22g, (0, 2, 1))
    xg_spec = pl.BlockSpec((1, NPG, H), lambda g: (g, 0, 0))
    z_spec = pl.BlockSpec((1, NPG, 8), lambda g: (g, 0, 0))
    zt_spec = pl.BlockSpec((1, 8, NPG), lambda g: (g, 0, 0))
    out_specs = []
    out_shape = []
    for _ in range(8):
        out_specs += [pl.BlockSpec((1, K, H), lambda g: (g, 0, 0)),
                      pl.BlockSpec((1, 1, K), lambda g: (g, 0, 0)),
                      pl.BlockSpec((1, 1, K), lambda g: (g, 0, 0))]
        out_shape += [jax.ShapeDtypeStruct((G, K, H), jnp.float32),
                      jax.ShapeDtypeStruct((G, 1, K), jnp.int32),
                      jax.ShapeDtypeStruct((G, 1, K), jnp.float32)]
    return pl.pallas_call(
        _pools_kernel,
        grid=(G,),
        in_specs=[xg_spec, xg_spec, z_spec, zt_spec, z_spec, zt_spec,
                  pl.BlockSpec((1, 8), lambda g: (0, 0))],
        out_specs=out_specs,
        out_shape=out_shape,
    )(x11g, x22g, z11g, z11t, z22g, z22t, norms)


def _head(f, W1, b1, W3, b3):
    KC = 4
    KCH = FLAT // KC
    return pl.pallas_call(
        _head_kernel,
        grid=(KC,),
        in_specs=[pl.BlockSpec((G, KCH), lambda k: (0, k)),
                  pl.BlockSpec((KCH, HC), lambda k: (k, 0)),
                  pl.BlockSpec((1, HC), lambda k: (0, 0)),
                  pl.BlockSpec((HC, 1), lambda k: (0, 0)),
                  pl.BlockSpec((1, 1), lambda k: (0, 0))],
        out_specs=pl.BlockSpec((G, 1), lambda k: (0, 0)),
        out_shape=jax.ShapeDtypeStruct((G, 1), jnp.float32),
        scratch_shapes=[pltpu.VMEM((G, HC), jnp.float32)],
    )(f, W1, b1.reshape(1, HC), W3, b3.reshape(1, 1))


def _conv_edges(h, hs, hd, src, dst):
    hss = hs.reshape(N)[src]
    hdd = hd.reshape(N)[dst]
    e = _edge_map(_edge_e_kernel, hss, hdd)
    m = jax.ops.segment_max(e, dst, num_segments=N)
    m = jnp.where(jnp.isfinite(m), m, 0.0)
    ex = _edge_map(_edge_ex_kernel, e, m[dst])
    s = jax.ops.segment_sum(ex, dst, num_segments=N)
    alpha = _edge_map(_edge_alpha_kernel, ex, s[dst])
    agg = jax.ops.segment_sum(alpha[:, None] * h[src], dst, num_segments=N)
    return agg


def kernel(x, edge_index, edge_weight, batch, params):
    del edge_weight, batch
    loops = jnp.arange(N, dtype=edge_index.dtype)
    src = jnp.concatenate([edge_index[0], loops])
    dst = jnp.concatenate([edge_index[1], loops])

    W0, as0, ad0, b0 = params['conv0']
    W1c, as1, ad1, b1c = params['conv1']
    W2c, as2, ad2, b2c = params['conv2']

    h0, hs0, hd0 = _conv0_mm(x, W0, as0, ad0)
    agg0 = _conv_edges(h0, hs0, hd0, src, dst)

    h1, hs1, hd1, h2, hs2, hd2 = _conv12_mm(agg0, b0, W1c, as1, ad1, W2c, as2, ad2)
    agg1 = _conv_edges(h1, hs1, hd1, src, dst)
    agg2 = _conv_edges(h2, hs2, hd2, src, dst)

    pools = jnp.stack(params['pools'], axis=1)
    x11, x22, z11, z22 = _final_act(agg1, b1c, agg2, b2c, pools)

    norms = jnp.stack([jnp.linalg.norm(p) + 1e-16 for p in params['pools']]
                      ).reshape(1, 8)

    pool_outs = _pools(x11, x22, z11, z22, norms)

    outs = []
    for i in range(8):
        xp = pool_outs[3 * i]
        perm = pool_outs[3 * i + 1]
        sig = pool_outs[3 * i + 2]
        W1, bb1, W3, bb3 = params['heads'][i]
        o = _head(xp.reshape(G, FLAT), W1, bb1, W3, bb3)
        outs += [o, perm.reshape(G * K), sig.reshape(G * K)]
    return tuple(outs)


# speedup vs baseline: 5.8310x; 5.8310x over previous
import functools

import jax
import jax.numpy as jnp
from jax import lax
from jax.experimental import pallas as pl
from jax.experimental.pallas import tpu as pltpu
from jax.experimental.pallas import tpu_sc as plsc

N = 33792
G = 128
NPG = 264
K = 132
H = 128
IN = 263
E = 270336
HC = 512
FLAT = K * H
EL = E + N
MB = 512
NBLK = N // MB


def _conv0_mm_kernel(x_ref, w_ref, as_ref, ad_ref, h_ref, hs_ref, hd_ref):
    h = jax.lax.dot_general(x_ref[...], w_ref[...], (((1,), (0,)), ((), ())),
                            preferred_element_type=jnp.float32)
    h_ref[...] = h
    hs_ref[...] = jax.lax.dot_general(h, as_ref[...], (((1,), (0,)), ((), ())),
                                      preferred_element_type=jnp.float32)
    hd_ref[...] = jax.lax.dot_general(h, ad_ref[...], (((1,), (0,)), ((), ())),
                                      preferred_element_type=jnp.float32)


def _conv12_mm_kernel(agg_ref, b_ref, w1_ref, as1_ref, ad1_ref,
                      w2_ref, as2_ref, ad2_ref,
                      h1_ref, hs1_ref, hd1_ref, h2_ref, hs2_ref, hd2_ref):
    x0 = jnp.maximum(agg_ref[...] + b_ref[...], 0.0)
    h1 = jax.lax.dot_general(x0, w1_ref[...], (((1,), (0,)), ((), ())),
                             preferred_element_type=jnp.float32)
    h1_ref[...] = h1
    hs1_ref[...] = jax.lax.dot_general(h1, as1_ref[...], (((1,), (0,)), ((), ())),
                                       preferred_element_type=jnp.float32)
    hd1_ref[...] = jax.lax.dot_general(h1, ad1_ref[...], (((1,), (0,)), ((), ())),
                                       preferred_element_type=jnp.float32)
    h2 = jax.lax.dot_general(x0, w2_ref[...], (((1,), (0,)), ((), ())),
                             preferred_element_type=jnp.float32)
    h2_ref[...] = h2
    hs2_ref[...] = jax.lax.dot_general(h2, as2_ref[...], (((1,), (0,)), ((), ())),
                                       preferred_element_type=jnp.float32)
    hd2_ref[...] = jax.lax.dot_general(h2, ad2_ref[...], (((1,), (0,)), ((), ())),
                                       preferred_element_type=jnp.float32)


def _final_act_kernel(agg1_ref, b1_ref, agg2_ref, b2_ref, p_ref,
                      x11_ref, x22_ref, z11_ref, z22_ref):
    x11 = jnp.maximum(agg1_ref[...] + b1_ref[...], 0.0)
    x22 = jnp.maximum(agg2_ref[...] + b2_ref[...], 0.0)
    x11_ref[...] = x11
    x22_ref[...] = x22
    for i in range(8):
        p_col = p_ref[:, i:i + 1]
        z11_ref[:, i:i + 1] = jax.lax.dot_general(
            x11, p_col, (((1,), (0,)), ((), ())), preferred_element_type=jnp.float32)
        z22_ref[:, i:i + 1] = jax.lax.dot_general(
            x22, p_col, (((1,), (0,)), ((), ())), preferred_element_type=jnp.float32)


def _edge_e_kernel(hss_ref, hdd_ref, e_ref):
    e_ref[...] = jax.nn.leaky_relu(hss_ref[...] + hdd_ref[...], 0.2)


def _edge_ex_kernel(e_ref, mdst_ref, ex_ref):
    ex_ref[...] = jnp.exp(e_ref[...] - mdst_ref[...])


def _edge_alpha_kernel(ex_ref, sdst_ref, a_ref):
    a_ref[...] = ex_ref[...] / (sdst_ref[...] + 1e-16)


def _pools_kernel(x11_ref, x22_ref, z11_ref, z11t_ref, z22_ref, z22t_ref, nrm_ref,
                  *out_refs):
    g = pl.program_id(0)
    kio_c = jax.lax.broadcasted_iota(jnp.int32, (K, 1), 0).astype(jnp.float32)
    kio_r = jax.lax.broadcasted_iota(jnp.int32, (1, K), 1).astype(jnp.float32)
    jio_c = jax.lax.broadcasted_iota(jnp.int32, (NPG, 1), 0).astype(jnp.float32)
    jio_r = jax.lax.broadcasted_iota(jnp.int32, (1, NPG), 1).astype(jnp.float32)
    ones_r = jnp.ones((1, NPG), jnp.float32)
    ones_c = jnp.ones((NPG, 1), jnp.float32)
    for i in range(8):
        if i < 4:
            xin = x11_ref[0]
            u_c = z11_ref[0, :, i:i + 1]
            u_r = z11t_ref[0, i:i + 1, :]
        else:
            xin = x22_ref[0]
            u_c = z22_ref[0, :, i:i + 1]
            u_r = z22t_ref[0, i:i + 1, :]
        d = nrm_ref[0:1, i:i + 1]
        s_c = jax.nn.sigmoid(u_c / d)
        s_r = jax.nn.sigmoid(u_r / d)
        cmp1 = jnp.where((s_c > s_r) | ((s_c == s_r) & (jio_c < jio_r)), 1.0, 0.0)
        rank_r = jax.lax.dot_general(ones_r, cmp1, (((1,), (0,)), ((), ())),
                                     preferred_element_type=jnp.float32)
        cmp2 = jnp.where((s_r > s_c) | ((s_r == s_c) & (jio_r < jio_c)), 1.0, 0.0)
        rank_c = jax.lax.dot_general(cmp2, ones_c, (((1,), (0,)), ((), ())),
                                     preferred_element_type=jnp.float32)
        onehot = jnp.where(kio_c == rank_r, 1.0, 0.0)
        onehot_t = jnp.where(rank_c == kio_r, 1.0, 0.0)
        idx_r = jax.lax.dot_general(jio_r, onehot_t, (((1,), (0,)), ((), ())),
                                    preferred_element_type=jnp.float32)
        vals_r = jax.lax.dot_general(s_r, onehot_t, (((1,), (0,)), ((), ())),
                                     preferred_element_type=jnp.float32)
        xp = jax.lax.dot_general(onehot, s_c * xin, (((1,), (0,)), ((), ())),
                                 preferred_element_type=jnp.float32)
        perm = (idx_r + (g * NPG).astype(jnp.float32)).astype(jnp.int32)
        sig = jax.nn.sigmoid(vals_r)
        out_refs[3 * i][...] = xp.reshape(1, K, H)
        out_refs[3 * i + 1][...] = perm.reshape(1, 1, K)
        out_refs[3 * i + 2][...] = sig.reshape(1, 1, K)


def _head_kernel(f_ref, w1_ref, b1_ref, w3_ref, b3_ref, o_ref, acc_ref):
    kc = pl.program_id(0)

    @pl.when(kc == 0)
    def _():
        acc_ref[...] = jnp.zeros_like(acc_ref)

    acc_ref[...] += jax.lax.dot_general(
        jnp.maximum(f_ref[...], 0.0), w1_ref[...], (((1,), (0,)), ((), ())),
        preferred_element_type=jnp.float32)

    @pl.when(kc == pl.num_programs(0) - 1)
    def _():
        h1 = jnp.maximum(acc_ref[...] + b1_ref[...], 0.0)
        o_ref[...] = jax.lax.dot_general(h1, w3_ref[...], (((1,), (0,)), ((), ())),
                                         preferred_element_type=jnp.float32) + b3_ref[...]


def _conv0_mm(x, W, a_s, a_d):
    return pl.pallas_call(
        _conv0_mm_kernel,
        grid=(NBLK,),
        in_specs=[pl.BlockSpec((MB, IN), lambda i: (i, 0)),
                  pl.BlockSpec((IN, H), lambda i: (0, 0)),
                  pl.BlockSpec((H, 1), lambda i: (0, 0)),
                  pl.BlockSpec((H, 1), lambda i: (0, 0))],
        out_specs=[pl.BlockSpec((MB, H), lambda i: (i, 0)),
                   pl.BlockSpec((MB, 1), lambda i: (i, 0)),
                   pl.BlockSpec((MB, 1), lambda i: (i, 0))],
        out_shape=[jax.ShapeDtypeStruct((N, H), jnp.float32),
                   jax.ShapeDtypeStruct((N, 1), jnp.float32),
                   jax.ShapeDtypeStruct((N, 1), jnp.float32)],
    )(x, W, a_s.reshape(H, 1), a_d.reshape(H, 1))


def _conv12_mm(agg0, b0, W1, as1, ad1, W2, as2, ad2):
    w_spec = pl.BlockSpec((H, H), lambda i: (0, 0))
    v_spec = pl.BlockSpec((H, 1), lambda i: (0, 0))
    m_spec = pl.BlockSpec((MB, H), lambda i: (i, 0))
    c_spec = pl.BlockSpec((MB, 1), lambda i: (i, 0))
    mat = jax.ShapeDtypeStruct((N, H), jnp.float32)
    col = jax.ShapeDtypeStruct((N, 1), jnp.float32)
    return pl.pallas_call(
        _conv12_mm_kernel,
        grid=(NBLK,),
        in_specs=[m_spec, pl.BlockSpec((1, H), lambda i: (0, 0)),
                  w_spec, v_spec, v_spec, w_spec, v_spec, v_spec],
        out_specs=[m_spec, c_spec, c_spec, m_spec, c_spec, c_spec],
        out_shape=[mat, col, col, mat, col, col],
    )(agg0, b0.reshape(1, H), W1, as1.reshape(H, 1), ad1.reshape(H, 1),
      W2, as2.reshape(H, 1), ad2.reshape(H, 1))


def _final_act(agg1, b1, agg2, b2, pools):
    m_spec = pl.BlockSpec((MB, H), lambda i: (i, 0))
    z_spec = pl.BlockSpec((MB, 8), lambda i: (i, 0))
    return pl.pallas_call(
        _final_act_kernel,
        grid=(NBLK,),
        in_specs=[m_spec, pl.BlockSpec((1, H), lambda i: (0, 0)),
                  m_spec, pl.BlockSpec((1, H), lambda i: (0, 0)),
                  pl.BlockSpec((H, 8), lambda i: (0, 0))],
        out_specs=[m_spec, m_spec, z_spec, z_spec],
        out_shape=[jax.ShapeDtypeStruct((N, H), jnp.float32),
                   jax.ShapeDtypeStruct((N, H), jnp.float32),
                   jax.ShapeDtypeStruct((N, 8), jnp.float32),
                   jax.ShapeDtypeStruct((N, 8), jnp.float32)],
    )(agg1, b1.reshape(1, H), agg2, b2.reshape(1, H), pools)


_EROWS = EL // 128


def _edge_map(kernel, *arrs):
    arrs2 = [a.reshape(_EROWS, 128) for a in arrs]
    out = pl.pallas_call(
        kernel,
        out_shape=jax.ShapeDtypeStruct((_EROWS, 128), jnp.float32),
    )(*arrs2)
    return out.reshape(EL)


def _pools(x11, x22, z11, z22, norms):
    x11g = x11.reshape(G, NPG, H)
    x22g = x22.reshape(G, NPG, H)
    z11g = z11.reshape(G, NPG, 8)
    z22g = z22.reshape(G, NPG, 8)
    z11t = jnp.transpose(z11g, (0, 2, 1))
    z22t = jnp.transpose(z22g, (0, 2, 1))
    xg_spec = pl.BlockSpec((1, NPG, H), lambda g: (g, 0, 0))
    z_spec = pl.BlockSpec((1, NPG, 8), lambda g: (g, 0, 0))
    zt_spec = pl.BlockSpec((1, 8, NPG), lambda g: (g, 0, 0))
    out_specs = []
    out_shape = []
    for _ in range(8):
        out_specs += [pl.BlockSpec((1, K, H), lambda g: (g, 0, 0)),
                      pl.BlockSpec((1, 1, K), lambda g: (g, 0, 0)),
                      pl.BlockSpec((1, 1, K), lambda g: (g, 0, 0))]
        out_shape += [jax.ShapeDtypeStruct((G, K, H), jnp.float32),
                      jax.ShapeDtypeStruct((G, 1, K), jnp.int32),
                      jax.ShapeDtypeStruct((G, 1, K), jnp.float32)]
    return pl.pallas_call(
        _pools_kernel,
        grid=(G,),
        in_specs=[xg_spec, xg_spec, z_spec, zt_spec, z_spec, zt_spec,
                  pl.BlockSpec((1, 8), lambda g: (0, 0))],
        out_specs=out_specs,
        out_shape=out_shape,
    )(x11g, x22g, z11g, z11t, z22g, z22t, norms)


def _head(f, W1, b1, W3, b3):
    KC = 4
    KCH = FLAT // KC
    return pl.pallas_call(
        _head_kernel,
        grid=(KC,),
        in_specs=[pl.BlockSpec((G, KCH), lambda k: (0, k)),
                  pl.BlockSpec((KCH, HC), lambda k: (k, 0)),
                  pl.BlockSpec((1, HC), lambda k: (0, 0)),
                  pl.BlockSpec((HC, 1), lambda k: (0, 0)),
                  pl.BlockSpec((1, 1), lambda k: (0, 0))],
        out_specs=pl.BlockSpec((G, 1), lambda k: (0, 0)),
        out_shape=jax.ShapeDtypeStruct((G, 1), jnp.float32),
        scratch_shapes=[pltpu.VMEM((G, HC), jnp.float32)],
    )(f, W1, b1.reshape(1, HC), W3, b3.reshape(1, 1))


_NW = 32
_WCH = EL // _NW


def _sc_gather2(table_a, idx_a, table_b, idx_b):
    mesh = plsc.VectorSubcoreMesh(core_axis_name="c", subcore_axis_name="s")

    @functools.partial(
        pl.kernel, mesh=mesh,
        out_type=[jax.ShapeDtypeStruct((EL,), jnp.float32),
                  jax.ShapeDtypeStruct((EL,), jnp.float32)],
        scratch_types=[pltpu.VMEM((_WCH,), jnp.int32),
                       pltpu.VMEM((_WCH,), jnp.float32),
                       pltpu.SemaphoreType.DMA],
    )
    def k(ta, ia, tb, ib, oa, ob, idx_v, val_v, sem):
        wid = lax.axis_index("s") * 2 + lax.axis_index("c")
        base = wid * _WCH
        pltpu.sync_copy(ia.at[pl.ds(base, _WCH)], idx_v)
        pltpu.async_copy(ta.at[idx_v], val_v, sem).wait()
        pltpu.sync_copy(val_v, oa.at[pl.ds(base, _WCH)])
        pltpu.sync_copy(ib.at[pl.ds(base, _WCH)], idx_v)
        pltpu.async_copy(tb.at[idx_v], val_v, sem).wait()
        pltpu.sync_copy(val_v, ob.at[pl.ds(base, _WCH)])

    return k(table_a, idx_a, table_b, idx_b)


def _sc_gather1(table, idx):
    mesh = plsc.VectorSubcoreMesh(core_axis_name="c", subcore_axis_name="s")

    @functools.partial(
        pl.kernel, mesh=mesh,
        out_type=jax.ShapeDtypeStruct((EL,), jnp.float32),
        scratch_types=[pltpu.VMEM((_WCH,), jnp.int32),
                       pltpu.VMEM((_WCH,), jnp.float32),
                       pltpu.SemaphoreType.DMA],
    )
    def k(t, i, o, idx_v, val_v, sem):
        wid = lax.axis_index("s") * 2 + lax.axis_index("c")
        base = wid * _WCH
        pltpu.sync_copy(i.at[pl.ds(base, _WCH)], idx_v)
        pltpu.async_copy(t.at[idx_v], val_v, sem).wait()
        pltpu.sync_copy(val_v, o.at[pl.ds(base, _WCH)])

    return k(table, idx)


def _conv_edges(h, hs, hd, src, dst):
    hss, hdd = _sc_gather2(hs.reshape(N), src, hd.reshape(N), dst)
    e = _edge_map(_edge_e_kernel, hss, hdd)
    m = jax.ops.segment_max(e, dst, num_segments=N)
    m = jnp.where(jnp.isfinite(m), m, 0.0)
    ex = _edge_map(_edge_ex_kernel, e, _sc_gather1(m, dst))
    s = jax.ops.segment_sum(ex, dst, num_segments=N)
    alpha = _edge_map(_edge_alpha_kernel, ex, _sc_gather1(s, dst))
    agg = jax.ops.segment_sum(alpha[:, None] * h[src], dst, num_segments=N)
    return agg


def kernel(x, edge_index, edge_weight, batch, params):
    del edge_weight, batch
    loops = jnp.arange(N, dtype=edge_index.dtype)
    src = jnp.concatenate([edge_index[0], loops])
    dst = jnp.concatenate([edge_index[1], loops])

    W0, as0, ad0, b0 = params['conv0']
    W1c, as1, ad1, b1c = params['conv1']
    W2c, as2, ad2, b2c = params['conv2']

    h0, hs0, hd0 = _conv0_mm(x, W0, as0, ad0)
    agg0 = _conv_edges(h0, hs0, hd0, src, dst)

    h1, hs1, hd1, h2, hs2, hd2 = _conv12_mm(agg0, b0, W1c, as1, ad1, W2c, as2, ad2)
    agg1 = _conv_edges(h1, hs1, hd1, src, dst)
    agg2 = _conv_edges(h2, hs2, hd2, src, dst)

    pools = jnp.stack(params['pools'], axis=1)
    x11, x22, z11, z22 = _final_act(agg1, b1c, agg2, b2c, pools)

    norms = jnp.stack([jnp.linalg.norm(p) + 1e-16 for p in params['pools']]
                      ).reshape(1, 8)

    pool_outs = _pools(x11, x22, z11, z22, norms)

    outs = []
    for i in range(8):
        xp = pool_outs[3 * i]
        perm = pool_outs[3 * i + 1]
        sig = pool_outs[3 * i + 2]
        W1, bb1, W3, bb3 = params['heads'][i]
        o = _head(xp.reshape(G, FLAT), W1, bb1, W3, bb3)
        outs += [o, perm.reshape(G * K), sig.reshape(G * K)]
    return tuple(outs)


# speedup vs baseline: 7.7768x; 1.3337x over previous
import functools

import jax
import jax.numpy as jnp
from jax import lax
from jax.experimental import pallas as pl
from jax.experimental.pallas import tpu as pltpu
from jax.experimental.pallas import tpu_sc as plsc

N = 33792
G = 128
NPG = 264
K = 132
H = 128
IN = 263
E = 270336
HC = 512
FLAT = K * H
EL = E + N
MB = 512
NBLK = N // MB


def _conv0_mm_kernel(x_ref, w_ref, as_ref, ad_ref, h_ref, hs_ref, hd_ref):
    h = jax.lax.dot_general(x_ref[...], w_ref[...], (((1,), (0,)), ((), ())),
                            preferred_element_type=jnp.float32)
    h_ref[...] = h
    hs_ref[...] = jax.lax.dot_general(h, as_ref[...], (((1,), (0,)), ((), ())),
                                      preferred_element_type=jnp.float32)
    hd_ref[...] = jax.lax.dot_general(h, ad_ref[...], (((1,), (0,)), ((), ())),
                                      preferred_element_type=jnp.float32)


def _conv12_mm_kernel(agg_ref, b_ref, w1_ref, as1_ref, ad1_ref,
                      w2_ref, as2_ref, ad2_ref,
                      h1_ref, hs1_ref, hd1_ref, h2_ref, hs2_ref, hd2_ref):
    x0 = jnp.maximum(agg_ref[...] + b_ref[...], 0.0)
    h1 = jax.lax.dot_general(x0, w1_ref[...], (((1,), (0,)), ((), ())),
                             preferred_element_type=jnp.float32)
    h1_ref[...] = h1
    hs1_ref[...] = jax.lax.dot_general(h1, as1_ref[...], (((1,), (0,)), ((), ())),
                                       preferred_element_type=jnp.float32)
    hd1_ref[...] = jax.lax.dot_general(h1, ad1_ref[...], (((1,), (0,)), ((), ())),
                                       preferred_element_type=jnp.float32)
    h2 = jax.lax.dot_general(x0, w2_ref[...], (((1,), (0,)), ((), ())),
                             preferred_element_type=jnp.float32)
    h2_ref[...] = h2
    hs2_ref[...] = jax.lax.dot_general(h2, as2_ref[...], (((1,), (0,)), ((), ())),
                                       preferred_element_type=jnp.float32)
    hd2_ref[...] = jax.lax.dot_general(h2, ad2_ref[...], (((1,), (0,)), ((), ())),
                                       preferred_element_type=jnp.float32)


def _final_act_kernel(agg1_ref, b1_ref, agg2_ref, b2_ref, p_ref,
                      x11_ref, x22_ref, z11_ref, z22_ref):
    x11 = jnp.maximum(agg1_ref[...] + b1_ref[...], 0.0)
    x22 = jnp.maximum(agg2_ref[...] + b2_ref[...], 0.0)
    x11_ref[...] = x11
    x22_ref[...] = x22
    for i in range(8):
        p_col = p_ref[:, i:i + 1]
        z11_ref[:, i:i + 1] = jax.lax.dot_general(
            x11, p_col, (((1,), (0,)), ((), ())), preferred_element_type=jnp.float32)
        z22_ref[:, i:i + 1] = jax.lax.dot_general(
            x22, p_col, (((1,), (0,)), ((), ())), preferred_element_type=jnp.float32)


def _edge_e_kernel(hss_ref, hdd_ref, e_ref):
    e_ref[...] = jax.nn.leaky_relu(hss_ref[...] + hdd_ref[...], 0.2)


def _edge_ex_kernel(e_ref, mdst_ref, ex_ref):
    ex_ref[...] = jnp.exp(e_ref[...] - mdst_ref[...])


def _edge_alpha_kernel(ex_ref, sdst_ref, a_ref):
    a_ref[...] = ex_ref[...] / (sdst_ref[...] + 1e-16)


def _pools_kernel(x11_ref, x22_ref, z11_ref, z11t_ref, z22_ref, z22t_ref, nrm_ref,
                  *out_refs):
    g = pl.program_id(0)
    kio_c = jax.lax.broadcasted_iota(jnp.int32, (K, 1), 0).astype(jnp.float32)
    kio_r = jax.lax.broadcasted_iota(jnp.int32, (1, K), 1).astype(jnp.float32)
    jio_c = jax.lax.broadcasted_iota(jnp.int32, (NPG, 1), 0).astype(jnp.float32)
    jio_r = jax.lax.broadcasted_iota(jnp.int32, (1, NPG), 1).astype(jnp.float32)
    ones_r = jnp.ones((1, NPG), jnp.float32)
    ones_c = jnp.ones((NPG, 1), jnp.float32)
    for i in range(8):
        if i < 4:
            xin = x11_ref[0]
            u_c = z11_ref[0, :, i:i + 1]
            u_r = z11t_ref[0, i:i + 1, :]
        else:
            xin = x22_ref[0]
            u_c = z22_ref[0, :, i:i + 1]
            u_r = z22t_ref[0, i:i + 1, :]
        d = nrm_ref[0:1, i:i + 1]
        s_c = jax.nn.sigmoid(u_c / d)
        s_r = jax.nn.sigmoid(u_r / d)
        cmp1 = jnp.where((s_c > s_r) | ((s_c == s_r) & (jio_c < jio_r)), 1.0, 0.0)
        rank_r = jax.lax.dot_general(ones_r, cmp1, (((1,), (0,)), ((), ())),
                                     preferred_element_type=jnp.float32)
        cmp2 = jnp.where((s_r > s_c) | ((s_r == s_c) & (jio_r < jio_c)), 1.0, 0.0)
        rank_c = jax.lax.dot_general(cmp2, ones_c, (((1,), (0,)), ((), ())),
                                     preferred_element_type=jnp.float32)
        onehot = jnp.where(kio_c == rank_r, 1.0, 0.0)
        onehot_t = jnp.where(rank_c == kio_r, 1.0, 0.0)
        idx_r = jax.lax.dot_general(jio_r, onehot_t, (((1,), (0,)), ((), ())),
                                    preferred_element_type=jnp.float32)
        vals_r = jax.lax.dot_general(s_r, onehot_t, (((1,), (0,)), ((), ())),
                                     preferred_element_type=jnp.float32)
        xp = jax.lax.dot_general(onehot, s_c * xin, (((1,), (0,)), ((), ())),
                                 preferred_element_type=jnp.float32)
        perm = (idx_r + (g * NPG).astype(jnp.float32)).astype(jnp.int32)
        sig = jax.nn.sigmoid(vals_r)
        out_refs[3 * i][...] = xp.reshape(1, K, H)
        out_refs[3 * i + 1][...] = perm.reshape(1, 1, K)
        out_refs[3 * i + 2][...] = sig.reshape(1, 1, K)


def _head_kernel(f_ref, w1_ref, b1_ref, w3_ref, b3_ref, o_ref, acc_ref):
    kc = pl.program_id(0)

    @pl.when(kc == 0)
    def _():
        acc_ref[...] = jnp.zeros_like(acc_ref)

    acc_ref[...] += jax.lax.dot_general(
        jnp.maximum(f_ref[...], 0.0), w1_ref[...], (((1,), (0,)), ((), ())),
        preferred_element_type=jnp.float32)

    @pl.when(kc == pl.num_programs(0) - 1)
    def _():
        h1 = jnp.maximum(acc_ref[...] + b1_ref[...], 0.0)
        o_ref[...] = jax.lax.dot_general(h1, w3_ref[...], (((1,), (0,)), ((), ())),
                                         preferred_element_type=jnp.float32) + b3_ref[...]


def _conv0_mm(x, W, a_s, a_d):
    return pl.pallas_call(
        _conv0_mm_kernel,
        grid=(NBLK,),
        in_specs=[pl.BlockSpec((MB, IN), lambda i: (i, 0)),
                  pl.BlockSpec((IN, H), lambda i: (0, 0)),
                  pl.BlockSpec((H, 1), lambda i: (0, 0)),
                  pl.BlockSpec((H, 1), lambda i: (0, 0))],
        out_specs=[pl.BlockSpec((MB, H), lambda i: (i, 0)),
                   pl.BlockSpec((MB, 1), lambda i: (i, 0)),
                   pl.BlockSpec((MB, 1), lambda i: (i, 0))],
        out_shape=[jax.ShapeDtypeStruct((N, H), jnp.float32),
                   jax.ShapeDtypeStruct((N, 1), jnp.float32),
                   jax.ShapeDtypeStruct((N, 1), jnp.float32)],
    )(x, W, a_s.reshape(H, 1), a_d.reshape(H, 1))


def _conv12_mm(agg0, b0, W1, as1, ad1, W2, as2, ad2):
    w_spec = pl.BlockSpec((H, H), lambda i: (0, 0))
    v_spec = pl.BlockSpec((H, 1), lambda i: (0, 0))
    m_spec = pl.BlockSpec((MB, H), lambda i: (i, 0))
    c_spec = pl.BlockSpec((MB, 1), lambda i: (i, 0))
    mat = jax.ShapeDtypeStruct((N, H), jnp.float32)
    col = jax.ShapeDtypeStruct((N, 1), jnp.float32)
    return pl.pallas_call(
        _conv12_mm_kernel,
        grid=(NBLK,),
        in_specs=[m_spec, pl.BlockSpec((1, H), lambda i: (0, 0)),
                  w_spec, v_spec, v_spec, w_spec, v_spec, v_spec],
        out_specs=[m_spec, c_spec, c_spec, m_spec, c_spec, c_spec],
        out_shape=[mat, col, col, mat, col, col],
    )(agg0, b0.reshape(1, H), W1, as1.reshape(H, 1), ad1.reshape(H, 1),
      W2, as2.reshape(H, 1), ad2.reshape(H, 1))


def _final_act(agg1, b1, agg2, b2, pools):
    m_spec = pl.BlockSpec((MB, H), lambda i: (i, 0))
    z_spec = pl.BlockSpec((MB, 8), lambda i: (i, 0))
    return pl.pallas_call(
        _final_act_kernel,
        grid=(NBLK,),
        in_specs=[m_spec, pl.BlockSpec((1, H), lambda i: (0, 0)),
                  m_spec, pl.BlockSpec((1, H), lambda i: (0, 0)),
                  pl.BlockSpec((H, 8), lambda i: (0, 0))],
        out_specs=[m_spec, m_spec, z_spec, z_spec],
        out_shape=[jax.ShapeDtypeStruct((N, H), jnp.float32),
                   jax.ShapeDtypeStruct((N, H), jnp.float32),
                   jax.ShapeDtypeStruct((N, 8), jnp.float32),
                   jax.ShapeDtypeStruct((N, 8), jnp.float32)],
    )(agg1, b1.reshape(1, H), agg2, b2.reshape(1, H), pools)


_EROWS = EL // 128


def _edge_map(kernel, *arrs):
    arrs2 = [a.reshape(_EROWS, 128) for a in arrs]
    out = pl.pallas_call(
        kernel,
        out_shape=jax.ShapeDtypeStruct((_EROWS, 128), jnp.float32),
    )(*arrs2)
    return out.reshape(EL)


def _pools(x11, x22, z11, z22, norms):
    x11g = x11.reshape(G, NPG, H)
    x22g = x22.reshape(G, NPG, H)
    z11g = z11.reshape(G, NPG, 8)
    z22g = z22.reshape(G, NPG, 8)
    z11t = jnp.transpose(z11g, (0, 2, 1))
    z22t = jnp.transpose(z22g, (0, 2, 1))
    xg_spec = pl.BlockSpec((1, NPG, H), lambda g: (g, 0, 0))
    z_spec = pl.BlockSpec((1, NPG, 8), lambda g: (g, 0, 0))
    zt_spec = pl.BlockSpec((1, 8, NPG), lambda g: (g, 0, 0))
    out_specs = []
    out_shape = []
    for _ in range(8):
        out_specs += [pl.BlockSpec((1, K, H), lambda g: (g, 0, 0)),
                      pl.BlockSpec((1, 1, K), lambda g: (g, 0, 0)),
                      pl.BlockSpec((1, 1, K), lambda g: (g, 0, 0))]
        out_shape += [jax.ShapeDtypeStruct((G, K, H), jnp.float32),
                      jax.ShapeDtypeStruct((G, 1, K), jnp.int32),
                      jax.ShapeDtypeStruct((G, 1, K), jnp.float32)]
    return pl.pallas_call(
        _pools_kernel,
        grid=(G,),
        in_specs=[xg_spec, xg_spec, z_spec, zt_spec, z_spec, zt_spec,
                  pl.BlockSpec((1, 8), lambda g: (0, 0))],
        out_specs=out_specs,
        out_shape=out_shape,
    )(x11g, x22g, z11g, z11t, z22g, z22t, norms)


def _head(f, W1, b1, W3, b3):
    KC = 4
    KCH = FLAT // KC
    return pl.pallas_call(
        _head_kernel,
        grid=(KC,),
        in_specs=[pl.BlockSpec((G, KCH), lambda k: (0, k)),
                  pl.BlockSpec((KCH, HC), lambda k: (k, 0)),
                  pl.BlockSpec((1, HC), lambda k: (0, 0)),
                  pl.BlockSpec((HC, 1), lambda k: (0, 0)),
                  pl.BlockSpec((1, 1), lambda k: (0, 0))],
        out_specs=pl.BlockSpec((G, 1), lambda k: (0, 0)),
        out_shape=jax.ShapeDtypeStruct((G, 1), jnp.float32),
        scratch_shapes=[pltpu.VMEM((G, HC), jnp.float32)],
    )(f, W1, b1.reshape(1, HC), W3, b3.reshape(1, 1))


_NW = 32
_WCH = EL // _NW


def _sc_gather2(table_a, idx_a, table_b, idx_b):
    mesh = plsc.VectorSubcoreMesh(core_axis_name="c", subcore_axis_name="s")

    @functools.partial(
        pl.kernel, mesh=mesh,
        out_type=[jax.ShapeDtypeStruct((EL,), jnp.float32),
                  jax.ShapeDtypeStruct((EL,), jnp.float32)],
        scratch_types=[pltpu.VMEM((_WCH,), jnp.int32),
                       pltpu.VMEM((_WCH,), jnp.float32),
                       pltpu.SemaphoreType.DMA],
    )
    def k(ta, ia, tb, ib, oa, ob, idx_v, val_v, sem):
        wid = lax.axis_index("s") * 2 + lax.axis_index("c")
        base = wid * _WCH
        pltpu.sync_copy(ia.at[pl.ds(base, _WCH)], idx_v)
        pltpu.async_copy(ta.at[idx_v], val_v, sem).wait()
        pltpu.sync_copy(val_v, oa.at[pl.ds(base, _WCH)])
        pltpu.sync_copy(ib.at[pl.ds(base, _WCH)], idx_v)
        pltpu.async_copy(tb.at[idx_v], val_v, sem).wait()
        pltpu.sync_copy(val_v, ob.at[pl.ds(base, _WCH)])

    return k(table_a, idx_a, table_b, idx_b)


def _sc_gather1(table, idx):
    mesh = plsc.VectorSubcoreMesh(core_axis_name="c", subcore_axis_name="s")

    @functools.partial(
        pl.kernel, mesh=mesh,
        out_type=jax.ShapeDtypeStruct((EL,), jnp.float32),
        scratch_types=[pltpu.VMEM((_WCH,), jnp.int32),
                       pltpu.VMEM((_WCH,), jnp.float32),
                       pltpu.SemaphoreType.DMA],
    )
    def k(t, i, o, idx_v, val_v, sem):
        wid = lax.axis_index("s") * 2 + lax.axis_index("c")
        base = wid * _WCH
        pltpu.sync_copy(i.at[pl.ds(base, _WCH)], idx_v)
        pltpu.async_copy(t.at[idx_v], val_v, sem).wait()
        pltpu.sync_copy(val_v, o.at[pl.ds(base, _WCH)])

    return k(table, idx)


_RCH = 528


def _sc_gather_rows(table, idx):
    mesh = plsc.VectorSubcoreMesh(core_axis_name="c", subcore_axis_name="s")

    @functools.partial(
        pl.kernel, mesh=mesh,
        out_type=jax.ShapeDtypeStruct((EL, H), jnp.float32),
        scratch_types=[pltpu.VMEM((_WCH,), jnp.int32),
                       pltpu.VMEM((_RCH, H), jnp.float32),
                       pltpu.SemaphoreType.DMA],
    )
    def k(t, i, o, idx_v, rows_v, sem):
        wid = lax.axis_index("s") * 2 + lax.axis_index("c")
        base = wid * _WCH
        pltpu.sync_copy(i.at[pl.ds(base, _WCH)], idx_v)
        for c in range(_WCH // _RCH):
            pltpu.async_copy(t.at[idx_v.at[pl.ds(c * _RCH, _RCH)]], rows_v, sem).wait()
            pltpu.sync_copy(rows_v, o.at[pl.ds(base + c * _RCH, _RCH)])

    return k(table, idx)


def _conv_edges(h, hs, hd, src, dst):
    hss, hdd = _sc_gather2(hs.reshape(N), src, hd.reshape(N), dst)
    e = _edge_map(_edge_e_kernel, hss, hdd)
    m = jax.ops.segment_max(e, dst, num_segments=N)
    m = jnp.where(jnp.isfinite(m), m, 0.0)
    ex = _edge_map(_edge_ex_kernel, e, _sc_gather1(m, dst))
    s = jax.ops.segment_sum(ex, dst, num_segments=N)
    alpha = _edge_map(_edge_alpha_kernel, ex, _sc_gather1(s, dst))
    hsrc = _sc_gather_rows(h, src)
    agg = jax.ops.segment_sum(alpha[:, None] * hsrc, dst, num_segments=N)
    return agg


def kernel(x, edge_index, edge_weight, batch, params):
    del edge_weight, batch
    loops = jnp.arange(N, dtype=edge_index.dtype)
    src = jnp.concatenate([edge_index[0], loops])
    dst = jnp.concatenate([edge_index[1], loops])

    W0, as0, ad0, b0 = params['conv0']
    W1c, as1, ad1, b1c = params['conv1']
    W2c, as2, ad2, b2c = params['conv2']

    h0, hs0, hd0 = _conv0_mm(x, W0, as0, ad0)
    agg0 = _conv_edges(h0, hs0, hd0, src, dst)

    h1, hs1, hd1, h2, hs2, hd2 = _conv12_mm(agg0, b0, W1c, as1, ad1, W2c, as2, ad2)
    agg1 = _conv_edges(h1, hs1, hd1, src, dst)
    agg2 = _conv_edges(h2, hs2, hd2, src, dst)

    pools = jnp.stack(params['pools'], axis=1)
    x11, x22, z11, z22 = _final_act(agg1, b1c, agg2, b2c, pools)

    norms = jnp.stack([jnp.linalg.norm(p) + 1e-16 for p in params['pools']]
                      ).reshape(1, 8)

    pool_outs = _pools(x11, x22, z11, z22, norms)

    outs = []
    for i in range(8):
        xp = pool_outs[3 * i]
        perm = pool_outs[3 * i + 1]
        sig = pool_outs[3 * i + 2]
        W1, bb1, W3, bb3 = params['heads'][i]
        o = _head(xp.reshape(G, FLAT), W1, bb1, W3, bb3)
        outs += [o, perm.reshape(G * K), sig.reshape(G * K)]
    return tuple(outs)


# speedup vs baseline: 8.5455x; 1.0988x over previous
import functools

import jax
import jax.numpy as jnp
from jax import lax
from jax.experimental import pallas as pl
from jax.experimental.pallas import tpu as pltpu
from jax.experimental.pallas import tpu_sc as plsc

N = 33792
G = 128
NPG = 264
K = 132
H = 128
IN = 263
E = 270336
HC = 512
FLAT = K * H
EL = E + N
MB = 512
NBLK = N // MB


def _conv0_mm_kernel(x_ref, w_ref, as_ref, ad_ref, h_ref, hs_ref, hd_ref):
    h = jax.lax.dot_general(x_ref[...], w_ref[...], (((1,), (0,)), ((), ())),
                            preferred_element_type=jnp.float32)
    h_ref[...] = h
    hs_ref[...] = jax.lax.dot_general(h, as_ref[...], (((1,), (0,)), ((), ())),
                                      preferred_element_type=jnp.float32)
    hd_ref[...] = jax.lax.dot_general(h, ad_ref[...], (((1,), (0,)), ((), ())),
                                      preferred_element_type=jnp.float32)


def _conv12_mm_kernel(agg_ref, b_ref, w1_ref, as1_ref, ad1_ref,
                      w2_ref, as2_ref, ad2_ref,
                      h1_ref, hs1_ref, hd1_ref, h2_ref, hs2_ref, hd2_ref):
    x0 = jnp.maximum(agg_ref[...] + b_ref[...], 0.0)
    h1 = jax.lax.dot_general(x0, w1_ref[...], (((1,), (0,)), ((), ())),
                             preferred_element_type=jnp.float32)
    h1_ref[...] = h1
    hs1_ref[...] = jax.lax.dot_general(h1, as1_ref[...], (((1,), (0,)), ((), ())),
                                       preferred_element_type=jnp.float32)
    hd1_ref[...] = jax.lax.dot_general(h1, ad1_ref[...], (((1,), (0,)), ((), ())),
                                       preferred_element_type=jnp.float32)
    h2 = jax.lax.dot_general(x0, w2_ref[...], (((1,), (0,)), ((), ())),
                             preferred_element_type=jnp.float32)
    h2_ref[...] = h2
    hs2_ref[...] = jax.lax.dot_general(h2, as2_ref[...], (((1,), (0,)), ((), ())),
                                       preferred_element_type=jnp.float32)
    hd2_ref[...] = jax.lax.dot_general(h2, ad2_ref[...], (((1,), (0,)), ((), ())),
                                       preferred_element_type=jnp.float32)


def _final_act_kernel(agg_ref, b_ref, p_ref, x_ref, z_ref):
    xv = jnp.maximum(agg_ref[...] + b_ref[...], 0.0)
    x_ref[...] = xv
    for i in range(4):
        p_col = p_ref[:, i:i + 1]
        z_ref[:, i:i + 1] = jax.lax.dot_general(
            xv, p_col, (((1,), (0,)), ((), ())), preferred_element_type=jnp.float32)


def _edge_e_kernel(hss_ref, hdd_ref, e_ref):
    e_ref[...] = jax.nn.leaky_relu(hss_ref[...] + hdd_ref[...], 0.2)


def _edge_ex_kernel(e_ref, mdst_ref, ex_ref):
    ex_ref[...] = jnp.exp(e_ref[...] - mdst_ref[...])


def _edge_alpha_kernel(ex_ref, sdst_ref, a_ref):
    a_ref[...] = ex_ref[...] / (sdst_ref[...] + 1e-16)


def _pools_kernel(x_ref, z_ref, zt_ref, nrm_ref, *out_refs):
    g = pl.program_id(0)
    kio_c = jax.lax.broadcasted_iota(jnp.int32, (K, 1), 0).astype(jnp.float32)
    jio_c = jax.lax.broadcasted_iota(jnp.int32, (NPG, 1), 0).astype(jnp.float32)
    jio_r = jax.lax.broadcasted_iota(jnp.int32, (1, NPG), 1).astype(jnp.float32)
    ones_r = jnp.ones((1, NPG), jnp.float32)
    xin = x_ref[0]
    for i in range(4):
        u_c = z_ref[0, :, i:i + 1]
        u_r = zt_ref[0, i:i + 1, :]
        d = nrm_ref[0:1, i:i + 1]
        s_c = jax.nn.sigmoid(u_c / d)
        s_r = jax.nn.sigmoid(u_r / d)
        cmp1 = jnp.where((s_c > s_r) | ((s_c == s_r) & (jio_c < jio_r)), 1.0, 0.0)
        rank_r = jax.lax.dot_general(ones_r, cmp1, (((1,), (0,)), ((), ())),
                                     preferred_element_type=jnp.float32)
        onehot = jnp.where(kio_c == rank_r, 1.0, 0.0)
        idx_c = jax.lax.dot_general(onehot, jio_c, (((1,), (0,)), ((), ())),
                                    preferred_element_type=jnp.float32)
        vals_c = jax.lax.dot_general(onehot, s_c, (((1,), (0,)), ((), ())),
                                     preferred_element_type=jnp.float32)
        xp = jax.lax.dot_general(onehot, s_c * xin, (((1,), (0,)), ((), ())),
                                 preferred_element_type=jnp.float32)
        perm = idx_c + (g * NPG).astype(jnp.float32)
        sig = jax.nn.sigmoid(vals_c)
        out_refs[3 * i][...] = xp.reshape(1, K, H)
        out_refs[3 * i + 1][...] = perm.reshape(1, K, 1)
        out_refs[3 * i + 2][...] = sig.reshape(1, K, 1)


def _head_kernel(f_ref, w1_ref, b1_ref, w3_ref, b3_ref, o_ref, acc_ref):
    kc = pl.program_id(0)

    @pl.when(kc == 0)
    def _():
        acc_ref[...] = jnp.zeros_like(acc_ref)

    acc_ref[...] += jax.lax.dot_general(
        jnp.maximum(f_ref[...], 0.0), w1_ref[...], (((1,), (0,)), ((), ())),
        preferred_element_type=jnp.float32)

    @pl.when(kc == pl.num_programs(0) - 1)
    def _():
        h1 = jnp.maximum(acc_ref[...] + b1_ref[...], 0.0)
        o_ref[...] = jax.lax.dot_general(h1, w3_ref[...], (((1,), (0,)), ((), ())),
                                         preferred_element_type=jnp.float32) + b3_ref[...]


def _conv0_mm(x, W, a_s, a_d):
    return pl.pallas_call(
        _conv0_mm_kernel,
        grid=(NBLK,),
        in_specs=[pl.BlockSpec((MB, IN), lambda i: (i, 0)),
                  pl.BlockSpec((IN, H), lambda i: (0, 0)),
                  pl.BlockSpec((H, 1), lambda i: (0, 0)),
                  pl.BlockSpec((H, 1), lambda i: (0, 0))],
        out_specs=[pl.BlockSpec((MB, H), lambda i: (i, 0)),
                   pl.BlockSpec((MB, 1), lambda i: (i, 0)),
                   pl.BlockSpec((MB, 1), lambda i: (i, 0))],
        out_shape=[jax.ShapeDtypeStruct((N, H), jnp.float32),
                   jax.ShapeDtypeStruct((N, 1), jnp.float32),
                   jax.ShapeDtypeStruct((N, 1), jnp.float32)],
    )(x, W, a_s.reshape(H, 1), a_d.reshape(H, 1))


def _conv12_mm(agg0, b0, W1, as1, ad1, W2, as2, ad2):
    w_spec = pl.BlockSpec((H, H), lambda i: (0, 0))
    v_spec = pl.BlockSpec((H, 1), lambda i: (0, 0))
    m_spec = pl.BlockSpec((MB, H), lambda i: (i, 0))
    c_spec = pl.BlockSpec((MB, 1), lambda i: (i, 0))
    mat = jax.ShapeDtypeStruct((N, H), jnp.float32)
    col = jax.ShapeDtypeStruct((N, 1), jnp.float32)
    return pl.pallas_call(
        _conv12_mm_kernel,
        grid=(NBLK,),
        in_specs=[m_spec, pl.BlockSpec((1, H), lambda i: (0, 0)),
                  w_spec, v_spec, v_spec, w_spec, v_spec, v_spec],
        out_specs=[m_spec, c_spec, c_spec, m_spec, c_spec, c_spec],
        out_shape=[mat, col, col, mat, col, col],
    )(agg0, b0.reshape(1, H), W1, as1.reshape(H, 1), ad1.reshape(H, 1),
      W2, as2.reshape(H, 1), ad2.reshape(H, 1))


def _final_act(agg, b, pools4):
    m_spec = pl.BlockSpec((MB, H), lambda i: (i, 0))
    z_spec = pl.BlockSpec((MB, 4), lambda i: (i, 0))
    return pl.pallas_call(
        _final_act_kernel,
        grid=(NBLK,),
        in_specs=[m_spec, pl.BlockSpec((1, H), lambda i: (0, 0)),
                  pl.BlockSpec((H, 4), lambda i: (0, 0))],
        out_specs=[m_spec, z_spec],
        out_shape=[jax.ShapeDtypeStruct((N, H), jnp.float32),
                   jax.ShapeDtypeStruct((N, 4), jnp.float32)],
    )(agg, b.reshape(1, H), pools4)


_EROWS = EL // 128


def _edge_map(kernel, *arrs):
    arrs2 = [a.reshape(_EROWS, 128) for a in arrs]
    out = pl.pallas_call(
        kernel,
        out_shape=jax.ShapeDtypeStruct((_EROWS, 128), jnp.float32),
    )(*arrs2)
    return out.reshape(EL)


def _pools(x, z, norms4):
    xg = x.reshape(G, NPG, H)
    zg = z.reshape(G, NPG, 4)
    zt = jnp.transpose(zg, (0, 2, 1))
    out_specs = []
    out_shape = []
    for _ in range(4):
        out_specs += [pl.BlockSpec((1, K, H), lambda g: (g, 0, 0)),
                      pl.BlockSpec((1, K, 1), lambda g: (g, 0, 0)),
                      pl.BlockSpec((1, K, 1), lambda g: (g, 0, 0))]
        out_shape += [jax.ShapeDtypeStruct((G, K, H), jnp.float32),
                      jax.ShapeDtypeStruct((G, K, 1), jnp.float32),
                      jax.ShapeDtypeStruct((G, K, 1), jnp.float32)]
    return pl.pallas_call(
        _pools_kernel,
        grid=(G,),
        in_specs=[pl.BlockSpec((1, NPG, H), lambda g: (g, 0, 0)),
                  pl.BlockSpec((1, NPG, 4), lambda g: (g, 0, 0)),
                  pl.BlockSpec((1, 4, NPG), lambda g: (g, 0, 0)),
                  pl.BlockSpec((1, 4), lambda g: (0, 0))],
        out_specs=out_specs,
        out_shape=out_shape,
    )(xg, zg, zt, norms4)


def _head(f, W1, b1, W3, b3):
    KC = 4
    KCH = FLAT // KC
    return pl.pallas_call(
        _head_kernel,
        grid=(KC,),
        in_specs=[pl.BlockSpec((G, KCH), lambda k: (0, k)),
                  pl.BlockSpec((KCH, HC), lambda k: (k, 0)),
                  pl.BlockSpec((1, HC), lambda k: (0, 0)),
                  pl.BlockSpec((HC, 1), lambda k: (0, 0)),
                  pl.BlockSpec((1, 1), lambda k: (0, 0))],
        out_specs=pl.BlockSpec((G, 1), lambda k: (0, 0)),
        out_shape=jax.ShapeDtypeStruct((G, 1), jnp.float32),
        scratch_shapes=[pltpu.VMEM((G, HC), jnp.float32)],
    )(f, W1, b1.reshape(1, HC), W3, b3.reshape(1, 1))


_NW = 32
_WCH = EL // _NW


def _sc_gather2(table_a, idx_a, table_b, idx_b):
    mesh = plsc.VectorSubcoreMesh(core_axis_name="c", subcore_axis_name="s")

    @functools.partial(
        pl.kernel, mesh=mesh,
        out_type=[jax.ShapeDtypeStruct((EL,), jnp.float32),
                  jax.ShapeDtypeStruct((EL,), jnp.float32)],
        scratch_types=[pltpu.VMEM((_WCH,), jnp.int32),
                       pltpu.VMEM((_WCH,), jnp.float32),
                       pltpu.SemaphoreType.DMA],
    )
    def k(ta, ia, tb, ib, oa, ob, idx_v, val_v, sem):
        wid = lax.axis_index("s") * 2 + lax.axis_index("c")
        base = wid * _WCH
        pltpu.sync_copy(ia.at[pl.ds(base, _WCH)], idx_v)
        pltpu.async_copy(ta.at[idx_v], val_v, sem).wait()
        pltpu.sync_copy(val_v, oa.at[pl.ds(base, _WCH)])
        pltpu.sync_copy(ib.at[pl.ds(base, _WCH)], idx_v)
        pltpu.async_copy(tb.at[idx_v], val_v, sem).wait()
        pltpu.sync_copy(val_v, ob.at[pl.ds(base, _WCH)])

    return k(table_a, idx_a, table_b, idx_b)


def _sc_gather1(table, idx):
    mesh = plsc.VectorSubcoreMesh(core_axis_name="c", subcore_axis_name="s")

    @functools.partial(
        pl.kernel, mesh=mesh,
        out_type=jax.ShapeDtypeStruct((EL,), jnp.float32),
        scratch_types=[pltpu.VMEM((_WCH,), jnp.int32),
                       pltpu.VMEM((_WCH,), jnp.float32),
                       pltpu.SemaphoreType.DMA],
    )
    def k(t, i, o, idx_v, val_v, sem):
        wid = lax.axis_index("s") * 2 + lax.axis_index("c")
        base = wid * _WCH
        pltpu.sync_copy(i.at[pl.ds(base, _WCH)], idx_v)
        pltpu.async_copy(t.at[idx_v], val_v, sem).wait()
        pltpu.sync_copy(val_v, o.at[pl.ds(base, _WCH)])

    return k(table, idx)


_RCH = 528


def _sc_gather_rows(table, idx):
    mesh = plsc.VectorSubcoreMesh(core_axis_name="c", subcore_axis_name="s")

    @functools.partial(
        pl.kernel, mesh=mesh,
        out_type=jax.ShapeDtypeStruct((EL, H), jnp.float32),
        scratch_types=[pltpu.VMEM((_WCH,), jnp.int32),
                       pltpu.VMEM((_RCH, H), jnp.float32),
                       pltpu.SemaphoreType.DMA],
    )
    def k(t, i, o, idx_v, rows_v, sem):
        wid = lax.axis_index("s") * 2 + lax.axis_index("c")
        base = wid * _WCH
        pltpu.sync_copy(i.at[pl.ds(base, _WCH)], idx_v)
        for c in range(_WCH // _RCH):
            pltpu.async_copy(t.at[idx_v.at[pl.ds(c * _RCH, _RCH)]], rows_v, sem).wait()
            pltpu.sync_copy(rows_v, o.at[pl.ds(base + c * _RCH, _RCH)])

    return k(table, idx)


def _conv_edges(h, hs, hd, src, dst):
    hss, hdd = _sc_gather2(hs.reshape(N), src, hd.reshape(N), dst)
    e = _edge_map(_edge_e_kernel, hss, hdd)
    m = jax.ops.segment_max(e, dst, num_segments=N)
    m = jnp.where(jnp.isfinite(m), m, 0.0)
    ex = _edge_map(_edge_ex_kernel, e, _sc_gather1(m, dst))
    s = jax.ops.segment_sum(ex, dst, num_segments=N)
    alpha = _edge_map(_edge_alpha_kernel, ex, _sc_gather1(s, dst))
    hsrc = _sc_gather_rows(h, src)
    agg = jax.ops.segment_sum(alpha[:, None] * hsrc, dst, num_segments=N)
    return agg


def kernel(x, edge_index, edge_weight, batch, params):
    del edge_weight, batch
    loops = jnp.arange(N, dtype=edge_index.dtype)
    src = jnp.concatenate([edge_index[0], loops])
    dst = jnp.concatenate([edge_index[1], loops])

    W0, as0, ad0, b0 = params['conv0']
    W1c, as1, ad1, b1c = params['conv1']
    W2c, as2, ad2, b2c = params['conv2']

    h0, hs0, hd0 = _conv0_mm(x, W0, as0, ad0)
    agg0 = _conv_edges(h0, hs0, hd0, src, dst)

    h1, hs1, hd1, h2, hs2, hd2 = _conv12_mm(agg0, b0, W1c, as1, ad1, W2c, as2, ad2)
    agg1 = _conv_edges(h1, hs1, hd1, src, dst)
    agg2 = _conv_edges(h2, hs2, hd2, src, dst)

    pools_a = jnp.stack(params['pools'][:4], axis=1)
    pools_b = jnp.stack(params['pools'][4:], axis=1)
    x11, z11 = _final_act(agg1, b1c, pools_a)
    x22, z22 = _final_act(agg2, b2c, pools_b)

    norms = [jnp.linalg.norm(p) + 1e-16 for p in params['pools']]
    norms_a = jnp.stack(norms[:4]).reshape(1, 4)
    norms_b = jnp.stack(norms[4:]).reshape(1, 4)

    pool_outs = list(_pools(x11, z11, norms_a)) + list(_pools(x22, z22, norms_b))

    outs = []
    for i in range(8):
        xp = pool_outs[3 * i]
        perm = pool_outs[3 * i + 1]
        sig = pool_outs[3 * i + 2]
        W1, bb1, W3, bb3 = params['heads'][i]
        o = _head(xp.reshape(G, FLAT), W1, bb1, W3, bb3)
        outs += [o, perm.reshape(G * K).astype(jnp.int32), sig.reshape(G * K)]
    return tuple(outs)


# speedup vs baseline: 8.5671x; 1.0025x over previous
import functools

import jax
import jax.numpy as jnp
from jax import lax
from jax.experimental import pallas as pl
from jax.experimental.pallas import tpu as pltpu
from jax.experimental.pallas import tpu_sc as plsc

N = 33792
G = 128
NPG = 264
K = 132
H = 128
IN = 263
E = 270336
HC = 512
FLAT = K * H
EL = E + N
MB = 512
NBLK = N // MB


def _conv0_mm_kernel(x_ref, w_ref, as_ref, ad_ref, h_ref, hs_ref, hd_ref):
    h = jax.lax.dot_general(x_ref[...], w_ref[...], (((1,), (0,)), ((), ())),
                            preferred_element_type=jnp.float32)
    h_ref[...] = h
    hs_ref[...] = jax.lax.dot_general(h, as_ref[...], (((1,), (0,)), ((), ())),
                                      preferred_element_type=jnp.float32)
    hd_ref[...] = jax.lax.dot_general(h, ad_ref[...], (((1,), (0,)), ((), ())),
                                      preferred_element_type=jnp.float32)


def _conv12_mm_kernel(agg_ref, b_ref, w1_ref, as1_ref, ad1_ref,
                      w2_ref, as2_ref, ad2_ref,
                      h1_ref, hs1_ref, hd1_ref, h2_ref, hs2_ref, hd2_ref):
    x0 = jnp.maximum(agg_ref[...] + b_ref[...], 0.0)
    h1 = jax.lax.dot_general(x0, w1_ref[...], (((1,), (0,)), ((), ())),
                             preferred_element_type=jnp.float32)
    h1_ref[...] = h1
    hs1_ref[...] = jax.lax.dot_general(h1, as1_ref[...], (((1,), (0,)), ((), ())),
                                       preferred_element_type=jnp.float32)
    hd1_ref[...] = jax.lax.dot_general(h1, ad1_ref[...], (((1,), (0,)), ((), ())),
                                       preferred_element_type=jnp.float32)
    h2 = jax.lax.dot_general(x0, w2_ref[...], (((1,), (0,)), ((), ())),
                             preferred_element_type=jnp.float32)
    h2_ref[...] = h2
    hs2_ref[...] = jax.lax.dot_general(h2, as2_ref[...], (((1,), (0,)), ((), ())),
                                       preferred_element_type=jnp.float32)
    hd2_ref[...] = jax.lax.dot_general(h2, ad2_ref[...], (((1,), (0,)), ((), ())),
                                       preferred_element_type=jnp.float32)


def _final_act_kernel(agg_ref, b_ref, p_ref, x_ref, z_ref):
    xv = jnp.maximum(agg_ref[...] + b_ref[...], 0.0)
    x_ref[...] = xv
    for i in range(4):
        p_col = p_ref[:, i:i + 1]
        z_ref[:, i:i + 1] = jax.lax.dot_general(
            xv, p_col, (((1,), (0,)), ((), ())), preferred_element_type=jnp.float32)


def _edge_e_kernel(hss_ref, hdd_ref, e_ref):
    e_ref[...] = jax.nn.leaky_relu(hss_ref[...] + hdd_ref[...], 0.2)


def _edge_ex_kernel(e_ref, mdst_ref, ex_ref):
    ex_ref[...] = jnp.exp(e_ref[...] - mdst_ref[...])


def _edge_alpha_kernel(ex_ref, sdst_ref, a_ref):
    a_ref[...] = ex_ref[...] / (sdst_ref[...] + 1e-16)


def _pools_kernel(x_ref, z_ref, zt_ref, nrm_ref, *out_refs):
    g = pl.program_id(0)
    kio_c = jax.lax.broadcasted_iota(jnp.int32, (K, 1), 0).astype(jnp.float32)
    jio_c = jax.lax.broadcasted_iota(jnp.int32, (NPG, 1), 0).astype(jnp.float32)
    jio_r = jax.lax.broadcasted_iota(jnp.int32, (1, NPG), 1).astype(jnp.float32)
    ones_r = jnp.ones((1, NPG), jnp.float32)
    xin = x_ref[0]
    for i in range(4):
        u_c = z_ref[0, :, i:i + 1]
        u_r = zt_ref[0, i:i + 1, :]
        d = nrm_ref[0:1, i:i + 1]
        s_c = jax.nn.sigmoid(u_c / d)
        s_r = jax.nn.sigmoid(u_r / d)
        cmp1 = jnp.where((s_c > s_r) | ((s_c == s_r) & (jio_c < jio_r)), 1.0, 0.0)
        rank_r = jax.lax.dot_general(ones_r, cmp1, (((1,), (0,)), ((), ())),
                                     preferred_element_type=jnp.float32)
        onehot = jnp.where(kio_c == rank_r, 1.0, 0.0)
        idx_c = jax.lax.dot_general(onehot, jio_c, (((1,), (0,)), ((), ())),
                                    preferred_element_type=jnp.float32)
        vals_c = jax.lax.dot_general(onehot, s_c, (((1,), (0,)), ((), ())),
                                     preferred_element_type=jnp.float32)
        xp = jax.lax.dot_general(onehot, s_c * xin, (((1,), (0,)), ((), ())),
                                 preferred_element_type=jnp.float32)
        perm = idx_c + (g * NPG).astype(jnp.float32)
        sig = jax.nn.sigmoid(vals_c)
        out_refs[3 * i][...] = xp.reshape(1, K, H)
        out_refs[3 * i + 1][...] = perm.reshape(1, K, 1)
        out_refs[3 * i + 2][...] = sig.reshape(1, K, 1)


def _head_kernel(f_ref, w1_ref, b1_ref, w3_ref, b3_ref, o_ref, acc_ref):
    kc = pl.program_id(0)

    @pl.when(kc == 0)
    def _():
        acc_ref[...] = jnp.zeros_like(acc_ref)

    acc_ref[...] += jax.lax.dot_general(
        jnp.maximum(f_ref[...], 0.0).astype(jnp.bfloat16), w1_ref[...],
        (((1,), (0,)), ((), ())), preferred_element_type=jnp.float32)

    @pl.when(kc == pl.num_programs(0) - 1)
    def _():
        h1 = jnp.maximum(acc_ref[...] + b1_ref[...], 0.0)
        o_ref[...] = jax.lax.dot_general(h1, w3_ref[...], (((1,), (0,)), ((), ())),
                                         preferred_element_type=jnp.float32) + b3_ref[...]


def _conv0_mm(x, W, a_s, a_d):
    return pl.pallas_call(
        _conv0_mm_kernel,
        grid=(NBLK,),
        in_specs=[pl.BlockSpec((MB, IN), lambda i: (i, 0)),
                  pl.BlockSpec((IN, H), lambda i: (0, 0)),
                  pl.BlockSpec((H, 1), lambda i: (0, 0)),
                  pl.BlockSpec((H, 1), lambda i: (0, 0))],
        out_specs=[pl.BlockSpec((MB, H), lambda i: (i, 0)),
                   pl.BlockSpec((MB, 1), lambda i: (i, 0)),
                   pl.BlockSpec((MB, 1), lambda i: (i, 0))],
        out_shape=[jax.ShapeDtypeStruct((N, H), jnp.float32),
                   jax.ShapeDtypeStruct((N, 1), jnp.float32),
                   jax.ShapeDtypeStruct((N, 1), jnp.float32)],
    )(x, W, a_s.reshape(H, 1), a_d.reshape(H, 1))


def _conv12_mm(agg0, b0, W1, as1, ad1, W2, as2, ad2):
    w_spec = pl.BlockSpec((H, H), lambda i: (0, 0))
    v_spec = pl.BlockSpec((H, 1), lambda i: (0, 0))
    m_spec = pl.BlockSpec((MB, H), lambda i: (i, 0))
    c_spec = pl.BlockSpec((MB, 1), lambda i: (i, 0))
    mat = jax.ShapeDtypeStruct((N, H), jnp.float32)
    col = jax.ShapeDtypeStruct((N, 1), jnp.float32)
    return pl.pallas_call(
        _conv12_mm_kernel,
        grid=(NBLK,),
        in_specs=[m_spec, pl.BlockSpec((1, H), lambda i: (0, 0)),
                  w_spec, v_spec, v_spec, w_spec, v_spec, v_spec],
        out_specs=[m_spec, c_spec, c_spec, m_spec, c_spec, c_spec],
        out_shape=[mat, col, col, mat, col, col],
    )(agg0, b0.reshape(1, H), W1, as1.reshape(H, 1), ad1.reshape(H, 1),
      W2, as2.reshape(H, 1), ad2.reshape(H, 1))


def _final_act(agg, b, pools4):
    m_spec = pl.BlockSpec((MB, H), lambda i: (i, 0))
    z_spec = pl.BlockSpec((MB, 4), lambda i: (i, 0))
    return pl.pallas_call(
        _final_act_kernel,
        grid=(NBLK,),
        in_specs=[m_spec, pl.BlockSpec((1, H), lambda i: (0, 0)),
                  pl.BlockSpec((H, 4), lambda i: (0, 0))],
        out_specs=[m_spec, z_spec],
        out_shape=[jax.ShapeDtypeStruct((N, H), jnp.float32),
                   jax.ShapeDtypeStruct((N, 4), jnp.float32)],
    )(agg, b.reshape(1, H), pools4)


_EROWS = EL // 128


def _edge_map(kernel, *arrs):
    arrs2 = [a.reshape(_EROWS, 128) for a in arrs]
    out = pl.pallas_call(
        kernel,
        out_shape=jax.ShapeDtypeStruct((_EROWS, 128), jnp.float32),
    )(*arrs2)
    return out.reshape(EL)


def _pools(x, z, norms4):
    xg = x.reshape(G, NPG, H)
    zg = z.reshape(G, NPG, 4)
    zt = jnp.transpose(zg, (0, 2, 1))
    out_specs = []
    out_shape = []
    for _ in range(4):
        out_specs += [pl.BlockSpec((1, K, H), lambda g: (g, 0, 0)),
                      pl.BlockSpec((1, K, 1), lambda g: (g, 0, 0)),
                      pl.BlockSpec((1, K, 1), lambda g: (g, 0, 0))]
        out_shape += [jax.ShapeDtypeStruct((G, K, H), jnp.float32),
                      jax.ShapeDtypeStruct((G, K, 1), jnp.float32),
                      jax.ShapeDtypeStruct((G, K, 1), jnp.float32)]
    return pl.pallas_call(
        _pools_kernel,
        grid=(G,),
        in_specs=[pl.BlockSpec((1, NPG, H), lambda g: (g, 0, 0)),
                  pl.BlockSpec((1, NPG, 4), lambda g: (g, 0, 0)),
                  pl.BlockSpec((1, 4, NPG), lambda g: (g, 0, 0)),
                  pl.BlockSpec((1, 4), lambda g: (0, 0))],
        out_specs=out_specs,
        out_shape=out_shape,
    )(xg, zg, zt, norms4)


def _head(f, W1, b1, W3, b3):
    KC = 4
    KCH = FLAT // KC
    return pl.pallas_call(
        _head_kernel,
        grid=(KC,),
        in_specs=[pl.BlockSpec((G, KCH), lambda k: (0, k)),
                  pl.BlockSpec((KCH, HC), lambda k: (k, 0)),
                  pl.BlockSpec((1, HC), lambda k: (0, 0)),
                  pl.BlockSpec((HC, 1), lambda k: (0, 0)),
                  pl.BlockSpec((1, 1), lambda k: (0, 0))],
        out_specs=pl.BlockSpec((G, 1), lambda k: (0, 0)),
        out_shape=jax.ShapeDtypeStruct((G, 1), jnp.float32),
        scratch_shapes=[pltpu.VMEM((G, HC), jnp.float32)],
    )(f, W1, b1.reshape(1, HC), W3, b3.reshape(1, 1))


_NW = 32
_WCH = EL // _NW


def _sc_gather2(table_a, idx_a, table_b, idx_b):
    mesh = plsc.VectorSubcoreMesh(core_axis_name="c", subcore_axis_name="s")

    @functools.partial(
        pl.kernel, mesh=mesh,
        out_type=[jax.ShapeDtypeStruct((EL,), jnp.float32),
                  jax.ShapeDtypeStruct((EL,), jnp.float32)],
        scratch_types=[pltpu.VMEM((_WCH,), jnp.int32),
                       pltpu.VMEM((_WCH,), jnp.float32),
                       pltpu.SemaphoreType.DMA],
    )
    def k(ta, ia, tb, ib, oa, ob, idx_v, val_v, sem):
        wid = lax.axis_index("s") * 2 + lax.axis_index("c")
        base = wid * _WCH
        pltpu.sync_copy(ia.at[pl.ds(base, _WCH)], idx_v)
        pltpu.async_copy(ta.at[idx_v], val_v, sem).wait()
        pltpu.sync_copy(val_v, oa.at[pl.ds(base, _WCH)])
        pltpu.sync_copy(ib.at[pl.ds(base, _WCH)], idx_v)
        pltpu.async_copy(tb.at[idx_v], val_v, sem).wait()
        pltpu.sync_copy(val_v, ob.at[pl.ds(base, _WCH)])

    return k(table_a, idx_a, table_b, idx_b)


def _sc_gather1(table, idx):
    mesh = plsc.VectorSubcoreMesh(core_axis_name="c", subcore_axis_name="s")

    @functools.partial(
        pl.kernel, mesh=mesh,
        out_type=jax.ShapeDtypeStruct((EL,), jnp.float32),
        scratch_types=[pltpu.VMEM((_WCH,), jnp.int32),
                       pltpu.VMEM((_WCH,), jnp.float32),
                       pltpu.SemaphoreType.DMA],
    )
    def k(t, i, o, idx_v, val_v, sem):
        wid = lax.axis_index("s") * 2 + lax.axis_index("c")
        base = wid * _WCH
        pltpu.sync_copy(i.at[pl.ds(base, _WCH)], idx_v)
        pltpu.async_copy(t.at[idx_v], val_v, sem).wait()
        pltpu.sync_copy(val_v, o.at[pl.ds(base, _WCH)])

    return k(table, idx)


_RCH = 528


def _sc_gather_rows(table, idx):
    mesh = plsc.VectorSubcoreMesh(core_axis_name="c", subcore_axis_name="s")

    @functools.partial(
        pl.kernel, mesh=mesh,
        out_type=jax.ShapeDtypeStruct((EL, H), jnp.float32),
        scratch_types=[pltpu.VMEM((_WCH,), jnp.int32),
                       pltpu.VMEM((_RCH, H), jnp.float32),
                       pltpu.SemaphoreType.DMA],
    )
    def k(t, i, o, idx_v, rows_v, sem):
        wid = lax.axis_index("s") * 2 + lax.axis_index("c")
        base = wid * _WCH
        pltpu.sync_copy(i.at[pl.ds(base, _WCH)], idx_v)
        for c in range(_WCH // _RCH):
            pltpu.async_copy(t.at[idx_v.at[pl.ds(c * _RCH, _RCH)]], rows_v, sem).wait()
            pltpu.sync_copy(rows_v, o.at[pl.ds(base + c * _RCH, _RCH)])

    return k(table, idx)


def _conv_edges(h, hs, hd, src, dst):
    hss, hdd = _sc_gather2(hs.reshape(N), src, hd.reshape(N), dst)
    e = _edge_map(_edge_e_kernel, hss, hdd)
    m = jax.ops.segment_max(e, dst, num_segments=N)
    m = jnp.where(jnp.isfinite(m), m, 0.0)
    ex = _edge_map(_edge_ex_kernel, e, _sc_gather1(m, dst))
    s = jax.ops.segment_sum(ex, dst, num_segments=N)
    alpha = _edge_map(_edge_alpha_kernel, ex, _sc_gather1(s, dst))
    hsrc = _sc_gather_rows(h, src)
    agg = jax.ops.segment_sum(alpha[:, None] * hsrc, dst, num_segments=N)
    return agg


def kernel(x, edge_index, edge_weight, batch, params):
    del edge_weight, batch
    loops = jnp.arange(N, dtype=edge_index.dtype)
    src = jnp.concatenate([edge_index[0], loops])
    dst = jnp.concatenate([edge_index[1], loops])

    W0, as0, ad0, b0 = params['conv0']
    W1c, as1, ad1, b1c = params['conv1']
    W2c, as2, ad2, b2c = params['conv2']

    h0, hs0, hd0 = _conv0_mm(x, W0, as0, ad0)
    agg0 = _conv_edges(h0, hs0, hd0, src, dst)

    h1, hs1, hd1, h2, hs2, hd2 = _conv12_mm(agg0, b0, W1c, as1, ad1, W2c, as2, ad2)
    agg1 = _conv_edges(h1, hs1, hd1, src, dst)
    agg2 = _conv_edges(h2, hs2, hd2, src, dst)

    pools_a = jnp.stack(params['pools'][:4], axis=1)
    pools_b = jnp.stack(params['pools'][4:], axis=1)
    x11, z11 = _final_act(agg1, b1c, pools_a)
    x22, z22 = _final_act(agg2, b2c, pools_b)

    norms = [jnp.linalg.norm(p) + 1e-16 for p in params['pools']]
    norms_a = jnp.stack(norms[:4]).reshape(1, 4)
    norms_b = jnp.stack(norms[4:]).reshape(1, 4)

    pool_outs = list(_pools(x11, z11, norms_a)) + list(_pools(x22, z22, norms_b))

    outs = []
    for i in range(8):
        xp = pool_outs[3 * i]
        perm = pool_outs[3 * i + 1]
        sig = pool_outs[3 * i + 2]
        W1, bb1, W3, bb3 = params['heads'][i]
        o = _head(xp.reshape(G, FLAT), W1.astype(jnp.bfloat16), bb1, W3, bb3)
        outs += [o, perm.reshape(G * K).astype(jnp.int32), sig.reshape(G * K)]
    return tuple(outs)
